# Initial kernel scaffold; baseline (speedup 1.0000x reference)
import jax
import jax.numpy as jnp
from jax.experimental import pallas as pl


def kernel(z_e, W):
    raise NotImplementedError("write your pallas kernel here")


# speedup vs baseline: 1.5264x; 1.5264x over previous
import jax
import jax.numpy as jnp
from jax.experimental import pallas as pl

_K = 1024
_D = 256


def _vq_body(z_ref, w_ref, zq_ref, idx_ref, loss_ref):
    b = pl.program_id(0)
    z = z_ref[0]
    w = w_ref[...]

    sumz = jnp.sum(z * z, axis=0, keepdims=True)
    sumw = jnp.sum(w * w, axis=1, keepdims=True)
    mm = jax.lax.dot_general(w, z, (((1,), (0,)), ((), ())),
                             preferred_element_type=jnp.float32)
    dmat = (sumz + sumw) - 2.0 * mm

    minval = jnp.min(dmat, axis=0, keepdims=True)
    kiota = jax.lax.broadcasted_iota(jnp.int32, dmat.shape, 0)
    idxv = jnp.min(jnp.where(dmat == minval, kiota, _K), axis=0)
    idx_ref[0, 0, :] = idxv

    onehot = (kiota == idxv[None, :]).astype(jnp.float32)
    zq = jax.lax.dot_general(w, onehot, (((0,), (0,)), ((), ())),
                             preferred_element_type=jnp.float32)
    zq_ref[0] = z + (zq - z)

    diff = z - zq
    part = jnp.sum(diff * diff, keepdims=True)

    @pl.when(b == 0)
    def _():
        loss_ref[...] = jnp.zeros_like(part)

    loss_ref[...] += part


def kernel(z_e, W):
    B, D, H, Wd = z_e.shape
    N = H * Wd
    z3 = z_e.reshape(B, D, N)

    zq3, idx3, loss_sum = pl.pallas_call(
        _vq_body,
        grid=(B,),
        in_specs=[
            pl.BlockSpec((1, D, N), lambda b: (b, 0, 0)),
            pl.BlockSpec((_K, D), lambda b: (0, 0)),
        ],
        out_specs=[
            pl.BlockSpec((1, D, N), lambda b: (b, 0, 0)),
            pl.BlockSpec((1, 1, N), lambda b: (b, 0, 0)),
            pl.BlockSpec((1, 1), lambda b: (0, 0)),
        ],
        out_shape=[
            jax.ShapeDtypeStruct((B, D, N), jnp.float32),
            jax.ShapeDtypeStruct((B, 1, N), jnp.int32),
            jax.ShapeDtypeStruct((1, 1), jnp.float32),
        ],
    )(z3, W)

    z_q_st = zq3.reshape(B, D, H, Wd)
    indices = idx3.reshape(B, H, Wd)
    loss = loss_sum[0, 0] / (B * D * N)
    return (z_q_st, loss, loss, indices)
